# baseline (speedup 1.0000x reference)
import functools

import jax
import jax.numpy as jnp
from jax import lax
from jax.experimental import pallas as pl
from jax.experimental.pallas import tpu as pltpu
from jax.experimental.pallas import tpu_sc as plsc

NC = 2
NS = 16
NW = NC * NS
LANES = 16
IDX_CHUNK = 128


def _sc_partials(label2d, feat, centers, b_per_w, d):
    n_chunks = b_per_w // IDX_CHUNK
    rows_per_w = b_per_w // IDX_CHUNK
    mesh = plsc.VectorSubcoreMesh(core_axis_name="c", subcore_axis_name="s")

    @functools.partial(
        pl.kernel,
        mesh=mesh,
        compiler_params=pltpu.CompilerParams(use_tc_tiling_on_sc=False),
        out_type=jax.ShapeDtypeStruct((NW, LANES), jnp.float32),
        scratch_types=[
            pltpu.VMEM((n_chunks, IDX_CHUNK), jnp.int32),
            pltpu.VMEM((b_per_w, d), jnp.float32),
            pltpu.VMEM((b_per_w, d), jnp.float32),
            pltpu.VMEM((LANES,), jnp.float32),
            pltpu.SemaphoreType.DMA,
            pltpu.SemaphoreType.DMA,
        ],
    )
    def k(label_hbm, feat_hbm, centers_hbm, out_hbm,
          idx_v, rows_v, feat_v, acc_v, gsem, fsem):
        wid = lax.axis_index("s") * NC + lax.axis_index("c")
        base = wid * b_per_w

        pltpu.sync_copy(label_hbm.at[pl.ds(wid * rows_per_w, rows_per_w)], idx_v)
        fcp = pltpu.async_copy(feat_hbm.at[pl.ds(base, b_per_w)], feat_v, fsem)
        copies = []
        for j in range(n_chunks):
            copies.append(pltpu.async_copy(
                centers_hbm.at[idx_v.at[j]],
                rows_v.at[pl.ds(j * IDX_CHUNK, IDX_CHUNK)],
                gsem))
        fcp.wait()
        for c in copies:
            c.wait()

        def body(i, acc):
            r0 = rows_v[i, pl.ds(0, LANES)]
            f0 = feat_v[i, pl.ds(0, LANES)]
            r1 = rows_v[i, pl.ds(LANES, LANES)]
            f1 = feat_v[i, pl.ds(LANES, LANES)]
            d0 = f0 - r0
            d1 = f1 - r1
            return acc + (d0 * d0 + d1 * d1)

        acc = lax.fori_loop(0, b_per_w, body, jnp.zeros((LANES,), jnp.float32))
        acc_v[...] = acc
        pltpu.sync_copy(acc_v, out_hbm.at[wid])

    return k(label2d, feat, centers)


def _tc_reduce(partials, scale):
    def body(x_ref, o_ref):
        o_ref[0, 0] = jnp.sum(x_ref[...]) * scale

    return pl.pallas_call(
        body,
        out_shape=jax.ShapeDtypeStruct((1, 1), jnp.float32),
        out_specs=pl.BlockSpec(memory_space=pltpu.SMEM),
    )(partials)


def kernel(label, feat, centers):
    b, d = feat.shape
    b_per_w = b // NW
    label2d = label.astype(jnp.int32).reshape(b // IDX_CHUNK, IDX_CHUNK)
    partials = _sc_partials(label2d, feat, centers, b_per_w, d)
    out = _tc_reduce(partials, 0.5 / b)
    return out.reshape(())
